# Initial kernel scaffold; baseline (speedup 1.0000x reference)
import jax
import jax.numpy as jnp
from jax.experimental import pallas as pl


def kernel(x, edge_index, W1, b1, W2, b2):
    raise NotImplementedError("write your pallas kernel here")


# speedup vs baseline: 8.2070x; 8.2070x over previous
import functools

import jax
import jax.numpy as jnp
from jax import lax
from jax.experimental import pallas as pl
from jax.experimental.pallas import tpu as pltpu
from jax.experimental.pallas import tpu_sc as plsc

N = 10000
NP = 10240
E = 160000
E_PAD = 163840
CHUNK = 128
NCH1 = 80
NCH2 = 40
NSUB = 16
STRIPE = NP // NSUB
BLK = 1024

_mesh = plsc.VectorSubcoreMesh(core_axis_name="c", subcore_axis_name="s")


@functools.partial(
    pl.kernel,
    out_type=jax.ShapeDtypeStruct((32, NP), jnp.float32),
    mesh=_mesh,
    scratch_types=[
        pltpu.VMEM((E_PAD // 32,), jnp.int32),
        pltpu.VMEM((NP,), jnp.float32),
    ],
    compiler_params=pltpu.CompilerParams(needs_layout_passes=False),
)
def _deg_kernel(dst_hbm, out_hbm, dstv, counts):
    c = lax.axis_index("c")
    s = lax.axis_index("s")
    w = c * NSUB + s

    def zero(i, carry):
        counts[pl.ds(i * 16, 16)] = jnp.zeros((16,), jnp.float32)
        return carry

    lax.fori_loop(0, NP // 16, zero, 0)
    pltpu.sync_copy(dst_hbm.at[w], dstv)
    ones = jnp.ones((16,), jnp.float32)

    def body(i, carry):
        idx = dstv[pl.ds(i * 16, 16)]
        plsc.addupdate_scatter(counts, [idx], ones)
        return carry

    lax.fori_loop(0, E_PAD // 32 // 16, body, 0)
    pltpu.sync_copy(counts, out_hbm.at[w])


@functools.partial(
    pl.kernel,
    out_type=[jax.ShapeDtypeStruct((NP, 128), jnp.float32),
              jax.ShapeDtypeStruct((NP, 128), jnp.float32)],
    mesh=_mesh,
    scratch_types=[
        pltpu.VMEM((NCH1, CHUNK), jnp.int32),
        pltpu.VMEM((NCH1, CHUNK), jnp.int32),
        pltpu.VMEM((CHUNK, 128), jnp.float32),
        pltpu.VMEM_SHARED((NP, 128), jnp.float32),
        pltpu.SemaphoreType.DMA,
    ],
)
def _seg1_kernel(ha_hbm, hb_hbm, src_hbm, dst_hbm, zero_hbm,
                 oa_hbm, ob_hbm, srcv, dstv, buf, acc, sem):
    c = lax.axis_index("c")
    s = lax.axis_index("s")
    pltpu.sync_copy(src_hbm.at[s], srcv)
    pltpu.sync_copy(dst_hbm.at[s], dstv)
    pltpu.sync_copy(zero_hbm.at[pl.ds(s * STRIPE, STRIPE)],
                    acc.at[pl.ds(s * STRIPE, STRIPE)])
    plsc.subcore_barrier()

    def run(h_hbm, out_hbm):
        def body(j, carry):
            pltpu.async_copy(h_hbm.at[srcv.at[j]], buf, sem).wait()
            pltpu.sync_copy(buf, acc.at[dstv.at[j]], add=True)
            return carry

        lax.fori_loop(0, NCH1, body, 0)
        plsc.subcore_barrier()
        pltpu.sync_copy(acc.at[pl.ds(s * STRIPE, STRIPE)],
                        out_hbm.at[pl.ds(s * STRIPE, STRIPE)])

    @pl.when(c == 0)
    def _():
        run(ha_hbm, oa_hbm)

    @pl.when(c == 1)
    def _():
        run(hb_hbm, ob_hbm)


@functools.partial(
    pl.kernel,
    out_type=[jax.ShapeDtypeStruct((NP, 64), jnp.float32),
              jax.ShapeDtypeStruct((NP, 64), jnp.float32)],
    mesh=_mesh,
    scratch_types=[
        pltpu.VMEM((NCH2, CHUNK), jnp.int32),
        pltpu.VMEM((NCH2, CHUNK), jnp.int32),
        pltpu.VMEM((CHUNK, 64), jnp.float32),
        pltpu.VMEM_SHARED((NP, 64), jnp.float32),
        pltpu.SemaphoreType.DMA,
    ],
    compiler_params=pltpu.CompilerParams(use_tc_tiling_on_sc=False),
)
def _seg2_kernel(h_hbm, src_hbm, dst_hbm, zero_hbm,
                 o0_hbm, o1_hbm, srcv, dstv, buf, acc, sem):
    c = lax.axis_index("c")
    s = lax.axis_index("s")
    w = c * NSUB + s
    pltpu.sync_copy(src_hbm.at[w], srcv)
    pltpu.sync_copy(dst_hbm.at[w], dstv)
    pltpu.sync_copy(zero_hbm.at[pl.ds(s * STRIPE, STRIPE)],
                    acc.at[pl.ds(s * STRIPE, STRIPE)])
    plsc.subcore_barrier()

    def body(j, carry):
        pltpu.async_copy(h_hbm.at[srcv.at[j]], buf, sem).wait()
        pltpu.sync_copy(buf, acc.at[dstv.at[j]], add=True)
        return carry

    lax.fori_loop(0, NCH2, body, 0)
    plsc.subcore_barrier()

    @pl.when(c == 0)
    def _():
        pltpu.sync_copy(acc.at[pl.ds(s * STRIPE, STRIPE)],
                        o0_hbm.at[pl.ds(s * STRIPE, STRIPE)])

    @pl.when(c == 1)
    def _():
        pltpu.sync_copy(acc.at[pl.ds(s * STRIPE, STRIPE)],
                        o1_hbm.at[pl.ds(s * STRIPE, STRIPE)])


def _dis_of(counts_blk):
    deg = jnp.sum(counts_blk, axis=0) + 1.0
    return lax.rsqrt(jnp.maximum(deg, 1.0))


def _tc1_body(counts_ref, x_ref, w1_ref, ha_ref, hb_ref):
    dis = _dis_of(counts_ref[...])
    h = jnp.dot(x_ref[...], w1_ref[...], preferred_element_type=jnp.float32)
    h = h * dis[:, None]
    ha_ref[...] = h[:, :128]
    hb_ref[...] = h[:, 128:]


def _tc2_body(counts_ref, s1a_ref, s1b_ref, ha_ref, hb_ref, b1_ref, w2_ref,
              out_ref):
    dis = _dis_of(counts_ref[...])
    agg = jnp.concatenate([s1a_ref[...] + ha_ref[...],
                           s1b_ref[...] + hb_ref[...]], axis=1)
    z = jnp.maximum(dis[:, None] * agg + b1_ref[...][0][None, :], 0.0)
    h2 = jnp.dot(z, w2_ref[...], preferred_element_type=jnp.float32)
    out_ref[...] = h2 * dis[:, None]


def _tc3_body(counts_ref, s0_ref, s1_ref, h2_ref, b2_ref, out_ref):
    dis = _dis_of(counts_ref[...])
    agg = s0_ref[...] + s1_ref[...] + h2_ref[...]
    out_ref[...] = dis[:, None] * agg + b2_ref[...][0][None, :]


def _counts_spec():
    return pl.BlockSpec((32, BLK), lambda i: (0, i))


def _row_spec(d):
    return pl.BlockSpec((BLK, d), lambda i: (i, 0))


def _full_spec(shape):
    return pl.BlockSpec(shape, lambda i: tuple(0 for _ in shape))


def _tc1(counts, xp, W1):
    return pl.pallas_call(
        _tc1_body,
        grid=(NP // BLK,),
        in_specs=[_counts_spec(), _row_spec(256), _full_spec((256, 256))],
        out_specs=[_row_spec(128), _row_spec(128)],
        out_shape=[jax.ShapeDtypeStruct((NP, 128), jnp.float32)] * 2,
    )(counts, xp, W1)


def _tc2(counts, s1a, s1b, ha, hb, b1t, W2):
    return pl.pallas_call(
        _tc2_body,
        grid=(NP // BLK,),
        in_specs=[_counts_spec(), _row_spec(128), _row_spec(128),
                  _row_spec(128), _row_spec(128), _full_spec((8, 256)),
                  _full_spec((256, 64))],
        out_specs=_row_spec(64),
        out_shape=jax.ShapeDtypeStruct((NP, 64), jnp.float32),
    )(counts, s1a, s1b, ha, hb, b1t, W2)


def _tc3(counts, s0, s1, h2s, b2t):
    return pl.pallas_call(
        _tc3_body,
        grid=(NP // BLK,),
        in_specs=[_counts_spec(), _row_spec(64), _row_spec(64), _row_spec(64),
                  _full_spec((8, 64))],
        out_specs=_row_spec(64),
        out_shape=jax.ShapeDtypeStruct((NP, 64), jnp.float32),
    )(counts, s0, s1, h2s, b2t)


def kernel(x, edge_index, W1, b1, W2, b2):
    n = x.shape[0]
    src = edge_index[0]
    dst = edge_index[1]
    pad = E_PAD - src.shape[0]
    sp = jnp.concatenate([src, jnp.zeros((pad,), jnp.int32)])
    dp = jnp.concatenate([dst, jnp.full((pad,), n, jnp.int32)])
    xp = jnp.pad(x, ((0, NP - n), (0, 0)))
    b1t = jnp.tile(b1[None, :], (8, 1))
    b2t = jnp.tile(b2[None, :], (8, 1))

    counts = _deg_kernel(dp.reshape(32, -1))
    ha, hb = _tc1(counts, xp, W1)
    s1a, s1b = _seg1_kernel(ha, hb,
                            sp.reshape(NSUB, NCH1, CHUNK),
                            dp.reshape(NSUB, NCH1, CHUNK),
                            jnp.zeros((NP, 128), jnp.float32))
    h2s = _tc2(counts, s1a, s1b, ha, hb, b1t, W2)
    s2p0, s2p1 = _seg2_kernel(h2s,
                              sp.reshape(32, NCH2, CHUNK),
                              dp.reshape(32, NCH2, CHUNK),
                              jnp.zeros((NP, 64), jnp.float32))
    out = _tc3(counts, s2p0, s2p1, h2s, b2t)
    return out[:n]
